# baseline (speedup 1.0000x reference)
import dataclasses
import functools

import jax
import jax.numpy as jnp
from jax import lax
from jax.experimental import pallas as pl
from jax.experimental.pallas import tpu as pltpu
from jax.experimental.pallas import tpu_sc as plsc

VOCAB = 50265
HIDDEN = 1024
PAD = 1
EPS = 1e-12

NC = 2
NS = 16
LANES = 16
NW = NC * NS
B, S = 4, 2048
TOKENS = B * S
TPW = TOKENS // NW
SEGS_PER_ROW = S // TPW
G = 16
NCHUNK = TPW // G


def _sc_gather_sum(input_ids, word_emb, pos_emb):
    mesh = plsc.VectorSubcoreMesh(core_axis_name="c", subcore_axis_name="s",
                                  num_cores=NC, num_subcores=NS)
    cp = pltpu.CompilerParams()
    if "needs_layout_passes" in pltpu.CompilerParams.__dataclass_fields__:
        cp = dataclasses.replace(cp, needs_layout_passes=False)

    @pl.kernel(
        compiler_params=cp,
        out_type=jax.ShapeDtypeStruct((TOKENS, HIDDEN), jnp.float32),
        mesh=mesh,
        scratch_types=[
            pltpu.VMEM((S,), jnp.int32),
            pltpu.VMEM((TPW,), jnp.int32),
            pltpu.VMEM((G, HIDDEN), jnp.float32),
            pltpu.VMEM((G, HIDDEN), jnp.float32),
            pltpu.VMEM((G, HIDDEN), jnp.float32),
            pltpu.VMEM((G, HIDDEN), jnp.float32),
            pltpu.SemaphoreType.DMA,
            pltpu.SemaphoreType.DMA,
            pltpu.SemaphoreType.DMA,
            pltpu.SemaphoreType.DMA,
            pltpu.SemaphoreType.DMA,
            pltpu.SemaphoreType.DMA,
        ],
    )
    def k(ids_hbm, word_hbm, pos_hbm, out_hbm, ids_v, pidx_v, wrows0, prows0,
          wrows1, prows1, wsem0, psem0, wsem1, psem1, osem0, osem1):
        wid = lax.axis_index("s") * NC + lax.axis_index("c")
        row = wid // SEGS_PER_ROW
        seg_off = (wid % SEGS_PER_ROW) * TPW
        base = wid * TPW

        pltpu.sync_copy(ids_hbm.at[row], ids_v)

        one = jnp.int32(1)
        zero = jnp.int32(0)

        def pre_body(i, acc):
            v = ids_v[pl.ds(i * LANES, LANES)]
            return acc + jnp.where(v != PAD, one, zero)

        acc = lax.fori_loop(0, seg_off // LANES, pre_body,
                            jnp.zeros((LANES,), jnp.int32))
        prefix = jnp.sum(acc)

        def pos_body(k_, carry):
            v = ids_v[pl.ds(seg_off + k_ * LANES, LANES)]
            m = jnp.where(v != PAD, one, zero)
            c = plsc.cumsum(m)
            pidx_v[pl.ds(k_ * LANES, LANES)] = (carry + c) * m + PAD
            return carry + jnp.sum(m)

        lax.fori_loop(0, TPW // LANES, pos_body, prefix)

        bufs = ((wrows0, prows0, wsem0, psem0, osem0),
                (wrows1, prows1, wsem1, psem1, osem1))

        def issue(g, s):
            wr, pr, ws, ps, _ = bufs[s]
            widx = ids_v.at[pl.ds(seg_off + g * G, G)]
            pidx = pidx_v.at[pl.ds(g * G, G)]
            return (pltpu.async_copy(word_hbm.at[widx], wr, ws),
                    pltpu.async_copy(pos_hbm.at[pidx], pr, ps))

        pending_gather = [issue(0, 0), None]
        pending_out = [None, None]
        for g in range(NCHUNK):
            s = g & 1
            ns = s ^ 1
            if g + 1 < NCHUNK:
                if pending_out[ns] is not None:
                    pending_out[ns].wait()
                pending_gather[ns] = issue(g + 1, ns)
            wc, pc = pending_gather[s]
            wc.wait()
            pc.wait()

            wr, pr, _, _, osem = bufs[s]

            @pl.loop(0, G)
            def _(r):
                @pl.loop(0, HIDDEN, step=LANES * 8)
                def _(c0):
                    for u in range(8):
                        sl = (r, pl.ds(c0 + u * LANES, LANES))
                        wr[sl] = wr[sl] + pr[sl]

            pending_out[s] = pltpu.async_copy(
                wr, out_hbm.at[pl.ds(base + g * G, G)], osem)

        pending_out[0].wait()
        pending_out[1].wait()

    return k(input_ids, word_emb, pos_emb)


def _ln_body(x_ref, t_ref, w_ref, b_ref, o_ref):
    x = x_ref[...] + t_ref[...]
    mu = jnp.mean(x, axis=-1, keepdims=True)
    d = x - mu
    var = jnp.mean(d * d, axis=-1, keepdims=True)
    o_ref[...] = d * lax.rsqrt(var + EPS) * w_ref[...] + b_ref[...]


def _tc_layernorm(summed, type_row, ln_w, ln_b):
    blk = 512
    return pl.pallas_call(
        _ln_body,
        grid=(TOKENS // blk,),
        in_specs=[
            pl.BlockSpec((blk, HIDDEN), lambda i: (i, 0)),
            pl.BlockSpec((1, HIDDEN), lambda i: (0, 0)),
            pl.BlockSpec((1, HIDDEN), lambda i: (0, 0)),
            pl.BlockSpec((1, HIDDEN), lambda i: (0, 0)),
        ],
        out_specs=pl.BlockSpec((blk, HIDDEN), lambda i: (i, 0)),
        out_shape=jax.ShapeDtypeStruct((TOKENS, HIDDEN), jnp.float32),
    )(summed, type_row, ln_w, ln_b)


def kernel(input_ids, word_emb, pos_emb, type_emb, ln_w, ln_b):
    summed = _sc_gather_sum(input_ids.astype(jnp.int32), word_emb, pos_emb)
    out = _tc_layernorm(summed, type_emb[0:1], ln_w.reshape(1, HIDDEN),
                        ln_b.reshape(1, HIDDEN))
    return out.reshape(B, S, HIDDEN)


# speedup vs baseline: 1.1454x; 1.1454x over previous
import dataclasses
import functools

import jax
import jax.numpy as jnp
from jax import lax
from jax.experimental import pallas as pl
from jax.experimental.pallas import tpu as pltpu
from jax.experimental.pallas import tpu_sc as plsc

VOCAB = 50265
HIDDEN = 1024
PAD = 1
EPS = 1e-12

NC = 2
NS = 16
LANES = 16
NW = NC * NS
B, S = 4, 2048
TOKENS = B * S
TPW = TOKENS // NW
SEGS_PER_ROW = S // TPW
G = 16
NCHUNK = TPW // G


def _sc_gather_sum(input_ids, word_emb, pos_emb):
    mesh = plsc.VectorSubcoreMesh(core_axis_name="c", subcore_axis_name="s",
                                  num_cores=NC, num_subcores=NS)
    cp = pltpu.CompilerParams()
    if "needs_layout_passes" in pltpu.CompilerParams.__dataclass_fields__:
        cp = dataclasses.replace(cp, needs_layout_passes=False)

    @pl.kernel(
        compiler_params=cp,
        out_type=jax.ShapeDtypeStruct((TOKENS, HIDDEN), jnp.float32),
        mesh=mesh,
        scratch_types=[
            pltpu.VMEM((S,), jnp.int32),
            pltpu.VMEM((TPW,), jnp.int32),
            pltpu.VMEM((G, HIDDEN), jnp.float32),
            pltpu.VMEM((G, HIDDEN), jnp.float32),
            pltpu.VMEM((G, HIDDEN), jnp.float32),
            pltpu.VMEM((G, HIDDEN), jnp.float32),
            pltpu.SemaphoreType.DMA,
            pltpu.SemaphoreType.DMA,
            pltpu.SemaphoreType.DMA,
            pltpu.SemaphoreType.DMA,
            pltpu.SemaphoreType.DMA,
            pltpu.SemaphoreType.DMA,
        ],
    )
    def k(ids_hbm, word_hbm, pos_hbm, out_hbm, ids_v, pidx_v, wrows0, prows0,
          wrows1, prows1, wsem0, psem0, wsem1, psem1, osem0, osem1):
        wid = lax.axis_index("s") * NC + lax.axis_index("c")
        row = wid // SEGS_PER_ROW
        seg_off = (wid % SEGS_PER_ROW) * TPW
        base = wid * TPW

        pltpu.sync_copy(ids_hbm.at[row], ids_v)

        one = jnp.int32(1)
        zero = jnp.int32(0)

        def pre_body(i, acc):
            v = ids_v[pl.ds(i * LANES, LANES)]
            return acc + jnp.where(v != PAD, one, zero)

        acc = lax.fori_loop(0, seg_off // LANES, pre_body,
                            jnp.zeros((LANES,), jnp.int32))
        prefix = jnp.sum(acc)

        def pos_body(k_, carry):
            v = ids_v[pl.ds(seg_off + k_ * LANES, LANES)]
            m = jnp.where(v != PAD, one, zero)
            c = plsc.cumsum(m)
            pidx_v[pl.ds(k_ * LANES, LANES)] = (carry + c) * m + PAD
            return carry + jnp.sum(m)

        lax.fori_loop(0, TPW // LANES, pos_body, prefix)

        bufs = ((wrows0, prows0, wsem0, psem0, osem0),
                (wrows1, prows1, wsem1, psem1, osem1))

        def issue(g, s):
            wr, pr, ws, ps, _ = bufs[s]
            widx = ids_v.at[pl.ds(seg_off + g * G, G)]
            pidx = pidx_v.at[pl.ds(g * G, G)]
            return (pltpu.async_copy(word_hbm.at[widx], wr, ws),
                    pltpu.async_copy(pos_hbm.at[pidx], pr, ps))

        pending_gather = [issue(0, 0), None]
        pending_out = [None, None]
        for g in range(NCHUNK):
            s = g & 1
            ns = s ^ 1
            if g + 1 < NCHUNK:
                if pending_out[ns] is not None:
                    pending_out[ns].wait()
                pending_gather[ns] = issue(g + 1, ns)
            wc, pc = pending_gather[s]
            wc.wait()
            pc.wait()

            wr, pr, _, _, osem = bufs[s]

            @pl.loop(0, G)
            def _(r):
                @pl.loop(0, HIDDEN, step=LANES)
                def _(c0):
                    sl = (r, pl.ds(c0, LANES))
                    wr[sl] = wr[sl] + pr[sl]

            pending_out[s] = pltpu.async_copy(
                wr, out_hbm.at[pl.ds(base + g * G, G)], osem)

        pending_out[0].wait()
        pending_out[1].wait()

    return k(input_ids, word_emb, pos_emb)


def _ln_body(x_ref, t_ref, w_ref, b_ref, o_ref):
    x = x_ref[...] + t_ref[...]
    mu = jnp.mean(x, axis=-1, keepdims=True)
    d = x - mu
    var = jnp.mean(d * d, axis=-1, keepdims=True)
    o_ref[...] = d * lax.rsqrt(var + EPS) * w_ref[...] + b_ref[...]


def _tc_layernorm(summed, type_row, ln_w, ln_b):
    blk = 512
    return pl.pallas_call(
        _ln_body,
        grid=(TOKENS // blk,),
        in_specs=[
            pl.BlockSpec((blk, HIDDEN), lambda i: (i, 0)),
            pl.BlockSpec((1, HIDDEN), lambda i: (0, 0)),
            pl.BlockSpec((1, HIDDEN), lambda i: (0, 0)),
            pl.BlockSpec((1, HIDDEN), lambda i: (0, 0)),
        ],
        out_specs=pl.BlockSpec((blk, HIDDEN), lambda i: (i, 0)),
        out_shape=jax.ShapeDtypeStruct((TOKENS, HIDDEN), jnp.float32),
    )(summed, type_row, ln_w, ln_b)


def kernel(input_ids, word_emb, pos_emb, type_emb, ln_w, ln_b):
    summed = _sc_gather_sum(input_ids.astype(jnp.int32), word_emb, pos_emb)
    out = _tc_layernorm(summed, type_emb[0:1], ln_w.reshape(1, HIDDEN),
                        ln_b.reshape(1, HIDDEN))
    return out.reshape(B, S, HIDDEN)


# speedup vs baseline: 1.5508x; 1.3540x over previous
import dataclasses
import functools

import jax
import jax.numpy as jnp
from jax import lax
from jax.experimental import pallas as pl
from jax.experimental.pallas import tpu as pltpu
from jax.experimental.pallas import tpu_sc as plsc

VOCAB = 50265
HIDDEN = 1024
PAD = 1
EPS = 1e-12

NC = 2
NS = 16
LANES = 16
NW = NC * NS
B, S = 4, 2048
TOKENS = B * S
TPW = TOKENS // NW
SEGS_PER_ROW = S // TPW
G = 16
NCHUNK = TPW // G
NBUF = 3


def _sc_gather(input_ids, word_emb, pos_emb):
    mesh = plsc.VectorSubcoreMesh(core_axis_name="c", subcore_axis_name="s",
                                  num_cores=NC, num_subcores=NS)
    cp = pltpu.CompilerParams()
    if "needs_layout_passes" in pltpu.CompilerParams.__dataclass_fields__:
        cp = dataclasses.replace(cp, needs_layout_passes=False)

    row_bufs = []
    sems = []
    for _ in range(NBUF):
        row_bufs += [pltpu.VMEM((G, HIDDEN), jnp.float32)] * 2
        sems += [pltpu.SemaphoreType.DMA] * 4

    @pl.kernel(
        compiler_params=cp,
        out_type=(jax.ShapeDtypeStruct((TOKENS, HIDDEN), jnp.float32),
                  jax.ShapeDtypeStruct((TOKENS, HIDDEN), jnp.float32)),
        mesh=mesh,
        scratch_types=[
            pltpu.VMEM((S,), jnp.int32),
            pltpu.VMEM((TPW,), jnp.int32),
        ] + row_bufs + sems,
    )
    def k(ids_hbm, word_hbm, pos_hbm, wout_hbm, pout_hbm, ids_v, pidx_v,
          *bufs_and_sems):
        wid = lax.axis_index("s") * NC + lax.axis_index("c")
        row = wid // SEGS_PER_ROW
        seg_off = (wid % SEGS_PER_ROW) * TPW
        base = wid * TPW

        bufs = []
        for i in range(NBUF):
            wr, pr = bufs_and_sems[2 * i:2 * i + 2]
            ws, ps, wos, pos_ = bufs_and_sems[2 * NBUF + 4 * i:
                                              2 * NBUF + 4 * i + 4]
            bufs.append((wr, pr, ws, ps, wos, pos_))

        pltpu.sync_copy(ids_hbm.at[row], ids_v)

        one = jnp.int32(1)
        zero = jnp.int32(0)

        def pre_body(i, acc):
            v = ids_v[pl.ds(i * LANES, LANES)]
            return acc + jnp.where(v != PAD, one, zero)

        acc = lax.fori_loop(0, seg_off // LANES, pre_body,
                            jnp.zeros((LANES,), jnp.int32))
        prefix = jnp.sum(acc)

        def pos_body(k_, carry):
            v = ids_v[pl.ds(seg_off + k_ * LANES, LANES)]
            m = jnp.where(v != PAD, one, zero)
            c = plsc.cumsum(m)
            pidx_v[pl.ds(k_ * LANES, LANES)] = (carry + c) * m + PAD
            return carry + jnp.sum(m)

        lax.fori_loop(0, TPW // LANES, pos_body, prefix)

        def issue_gathers(g, s):
            wr, pr, ws, ps, _, _ = bufs[s]
            widx = ids_v.at[pl.ds(seg_off + g * G, G)]
            pidx = pidx_v.at[pl.ds(g * G, G)]
            return (pltpu.async_copy(word_hbm.at[widx], wr, ws),
                    pltpu.async_copy(pos_hbm.at[pidx], pr, ps))

        pending = [issue_gathers(s, s) for s in range(NBUF)]
        pending_out = [None] * NBUF
        for g in range(NCHUNK):
            s = g % NBUF
            wc, pc = pending[s]
            wc.wait()
            pc.wait()
            wr, pr, _, _, wos, pos_ = bufs[s]
            dst = pl.ds(base + g * G, G)
            pending_out[s] = (
                pltpu.async_copy(wr, wout_hbm.at[dst], wos),
                pltpu.async_copy(pr, pout_hbm.at[dst], pos_))
            if g + NBUF < NCHUNK:
                oc, oc2 = pending_out[s]
                oc.wait()
                oc2.wait()
                pending[s] = issue_gathers(g + NBUF, s)
                pending_out[s] = None

        for s in range(NBUF):
            if pending_out[s] is not None:
                oc, oc2 = pending_out[s]
                oc.wait()
                oc2.wait()

    return k(input_ids, word_emb, pos_emb)


def _ln_body(w_ref, p_ref, t_ref, g_ref, b_ref, o_ref):
    x = w_ref[...] + p_ref[...] + t_ref[...]
    mu = jnp.mean(x, axis=-1, keepdims=True)
    d = x - mu
    var = jnp.mean(d * d, axis=-1, keepdims=True)
    o_ref[...] = d * lax.rsqrt(var + EPS) * g_ref[...] + b_ref[...]


def _tc_layernorm(wrows, prows, type_row, ln_w, ln_b):
    blk = 512
    return pl.pallas_call(
        _ln_body,
        grid=(TOKENS // blk,),
        in_specs=[
            pl.BlockSpec((blk, HIDDEN), lambda i: (i, 0)),
            pl.BlockSpec((blk, HIDDEN), lambda i: (i, 0)),
            pl.BlockSpec((1, HIDDEN), lambda i: (0, 0)),
            pl.BlockSpec((1, HIDDEN), lambda i: (0, 0)),
            pl.BlockSpec((1, HIDDEN), lambda i: (0, 0)),
        ],
        out_specs=pl.BlockSpec((blk, HIDDEN), lambda i: (i, 0)),
        out_shape=jax.ShapeDtypeStruct((TOKENS, HIDDEN), jnp.float32),
    )(wrows, prows, type_row, ln_w, ln_b)


def kernel(input_ids, word_emb, pos_emb, type_emb, ln_w, ln_b):
    wrows, prows = _sc_gather(input_ids.astype(jnp.int32), word_emb, pos_emb)
    out = _tc_layernorm(wrows, prows, type_emb[0:1], ln_w.reshape(1, HIDDEN),
                        ln_b.reshape(1, HIDDEN))
    return out.reshape(B, S, HIDDEN)
